# baseline (speedup 1.0000x reference)
import functools

import jax
import jax.numpy as jnp
from jax import lax
from jax.experimental import pallas as pl
from jax.experimental.pallas import tpu as pltpu
from jax.experimental.pallas import tpu_sc as plsc

_B = 16384
_D = 128
_NW = 32
_BPW = _B // _NW
_CH = 64
_NCH = _BPW // _CH
_GL = 16
_PK = _D // _GL


def _sc_gather_gmf(uidx2, iidx2, mu, mi, gu, gi, wg):
    mesh = plsc.VectorSubcoreMesh(core_axis_name="c", subcore_axis_name="s")
    f32 = jnp.float32

    @functools.partial(
        pl.kernel,
        out_type=(
            jax.ShapeDtypeStruct((_B, _D), f32),
            jax.ShapeDtypeStruct((_B, _D), f32),
            jax.ShapeDtypeStruct((_B // _PK, _D), f32),
        ),
        mesh=mesh,
        scratch_types=(
            pltpu.VMEM((_NCH, _CH), jnp.int32),
            pltpu.VMEM((_NCH, _CH), jnp.int32),
            pltpu.VMEM((_D,), f32),
            [pltpu.VMEM((_CH, _D), f32)] * 4,
            [pltpu.VMEM((_CH, _D), f32)] * 4,
            [pltpu.VMEM((_CH, _D), f32)] * 3,
            [pltpu.VMEM((_CH, _D), f32)] * 3,
            pltpu.VMEM((_BPW // _PK, _D), f32),
            [pltpu.SemaphoreType.DMA] * 3,
            [pltpu.SemaphoreType.DMA] * 4,
        ),
    )
    def run(uidx_h, iidx_h, mu_h, mi_h, gu_h, gi_h, wg_h,
            out_mu, out_mi, out_g,
            uidx_v, iidx_v, wg_v, mu_s, mi_s, gu_s, gi_s, gacc, gsem, wsem):
        wid = lax.axis_index("s") * 2 + lax.axis_index("c")
        pltpu.sync_copy(uidx_h.at[pl.ds(wid * _NCH, _NCH)], uidx_v)
        pltpu.sync_copy(iidx_h.at[pl.ds(wid * _NCH, _NCH)], iidx_v)

        g_desc = [None, None, None]
        w_desc = [None, None, None, None]

        def fire_gathers(j):
            s = j % 3
            g_desc[s] = [
                pltpu.async_copy(mu_h.at[uidx_v.at[j]], mu_s[j % 4], gsem[s]),
                pltpu.async_copy(mi_h.at[iidx_v.at[j]], mi_s[j % 4], gsem[s]),
                pltpu.async_copy(gu_h.at[uidx_v.at[j]], gu_s[s], gsem[s]),
                pltpu.async_copy(gi_h.at[iidx_v.at[j]], gi_s[s], gsem[s]),
            ]

        fire_gathers(0)
        fire_gathers(1)
        pltpu.sync_copy(wg_h.at[0, pl.ds(0, _D)], wg_v)
        for j in range(_NCH):
            ms = j % 3
            if j + 2 < _NCH:
                if j >= 2:
                    for d in w_desc[(j + 2) % 4]:
                        d.wait()
                fire_gathers(j + 2)
            for d in g_desc[ms]:
                d.wait()

            gur, gir = gu_s[ms], gi_s[ms]

            def gmf_group(g, _):
                row = g * _PK
                for k in range(_PK):
                    for c in range(_PK):
                        sl = pl.ds(c * _GL, _GL)
                        p = gur[row + k, sl] * gir[row + k, sl] * wg_v[sl]
                        acc = p if c == 0 else acc + p
                    gacc[j * (_CH // _PK) + g, pl.ds(k * _GL, _GL)] = acc
                return 0

            lax.fori_loop(0, _CH // _PK, gmf_group, 0)

            ob = wid * _BPW + j * _CH
            w_desc[j % 4] = [
                pltpu.async_copy(mu_s[j % 4], out_mu.at[pl.ds(ob, _CH)],
                                 wsem[j % 4]),
                pltpu.async_copy(mi_s[j % 4], out_mi.at[pl.ds(ob, _CH)],
                                 wsem[j % 4]),
            ]

        for j in range(_NCH - 4, _NCH):
            for d in w_desc[j % 4]:
                d.wait()
        pltpu.sync_copy(gacc, out_g.at[pl.ds(wid * (_BPW // _PK), _BPW // _PK)])

    return run(uidx2, iidx2, mu, mi, gu, gi, wg)


_BLK = 4096


def _tc_dense(um, im, gp, W1, b1, W2, b2, W3, b3, W_last, blast):
    f32 = jnp.float32

    def body(um_r, im_r, g_r, w1_r, b1_r, w2_r, b2_r, w3_r, b3_r,
             wl_r, bl_r, st_r, p_r, m_r, out_r):
        dot = functools.partial(lax.dot_general,
                                dimension_numbers=(((1,), (1,)), ((), ())),
                                preferred_element_type=f32)
        w1 = w1_r[...]
        h = jnp.maximum(dot(um_r[...], w1[:, :_D]) + dot(im_r[...], w1[:, _D:])
                        + b1_r[...].reshape(1, 64), 0.0)
        h = jnp.maximum(dot(h, w2_r[...]) + b2_r[...].reshape(1, 16), 0.0)
        h = jnp.maximum(dot(h, w3_r[...]) + b3_r[...].reshape(1, 8), 0.0)

        rs = dot(g_r[...], st_r[...])
        ps = lax.dot_general(p_r[...], rs, (((1,), (0,)), ((), ())),
                             preferred_element_type=f32)
        g = jnp.sum(ps * m_r[...], axis=1, keepdims=True)

        wm = wl_r[...][:, _D:]
        out_r[...] = (g + jnp.sum(h * wm, axis=1, keepdims=True)
                      + bl_r[...].reshape(1, 1))

    iota = functools.partial(lax.broadcasted_iota, jnp.int32)
    bp = _BLK // _PK
    St = (iota((_PK, _D), 1) // _GL == iota((_PK, _D), 0)).astype(f32)
    P = (iota((_BLK, bp), 0) // _PK == iota((_BLK, bp), 1)).astype(f32)
    M = (iota((_BLK, _PK), 0) % _PK == iota((_BLK, _PK), 1)).astype(f32)

    full = lambda shape: pl.BlockSpec(shape, lambda i: (0,) * len(shape))
    batch = pl.BlockSpec((_BLK, _D), lambda i: (i, 0))
    return pl.pallas_call(
        body,
        grid=(_B // _BLK,),
        in_specs=[batch, batch,
                  pl.BlockSpec((_BLK // _PK, _D), lambda i: (i, 0)),
                  full((64, 2 * _D)), full((64,)),
                  full((16, 64)), full((16,)),
                  full((8, 16)), full((8,)),
                  full((1, _D + 8)), full((1,)),
                  full((_PK, _D)), full((_BLK, bp)), full((_BLK, _PK))],
        out_specs=pl.BlockSpec((_BLK, 1), lambda i: (i, 0)),
        out_shape=jax.ShapeDtypeStruct((_B, 1), f32),
    )(um, im, gp, W1, b1, W2, b2, W3, b3, W_last, blast, St, P, M)


def kernel(x, mlp_user_w, mlp_item_w, gmf_user_w, gmf_item_w,
           W1, b1, W2, b2, W3, b3, W_last, b_last):
    uidx2 = x[:, 0].reshape(_B // _CH, _CH)
    iidx2 = x[:, 1].reshape(_B // _CH, _CH)
    um, im, gp = _sc_gather_gmf(uidx2, iidx2,
                                mlp_user_w, mlp_item_w,
                                gmf_user_w, gmf_item_w, W_last)
    out = _tc_dense(um, im, gp, W1, b1, W2, b2, W3, b3, W_last, b_last)
    return out


# speedup vs baseline: 1.0598x; 1.0598x over previous
import functools

import jax
import jax.numpy as jnp
from jax import lax
from jax.experimental import pallas as pl
from jax.experimental.pallas import tpu as pltpu
from jax.experimental.pallas import tpu_sc as plsc

_B = 16384
_D = 128
_NW = 32
_BPW = _B // _NW
_CH = 64
_NCH = _BPW // _CH
_GL = 16
_PK = _D // _GL


def _sc_gather_gmf(uidx2, iidx2, mu, mi, gu, gi, wg):
    mesh = plsc.VectorSubcoreMesh(core_axis_name="c", subcore_axis_name="s")
    f32 = jnp.float32

    @functools.partial(
        pl.kernel,
        out_type=(
            jax.ShapeDtypeStruct((_B, _D), f32),
            jax.ShapeDtypeStruct((_B, _D), f32),
            jax.ShapeDtypeStruct((_B // _PK, _D), f32),
        ),
        mesh=mesh,
        scratch_types=(
            pltpu.VMEM((_NCH, _CH), jnp.int32),
            pltpu.VMEM((_NCH, _CH), jnp.int32),
            pltpu.VMEM((_D,), f32),
            [pltpu.VMEM((_CH, _D), f32)] * 4,
            [pltpu.VMEM((_CH, _D), f32)] * 4,
            [pltpu.VMEM((_CH, _D), f32)] * 3,
            [pltpu.VMEM((_CH, _D), f32)] * 3,
            pltpu.VMEM((_BPW // _PK, _D), f32),
            [pltpu.SemaphoreType.DMA] * 3,
            [pltpu.SemaphoreType.DMA] * 4,
        ),
    )
    def run(uidx_h, iidx_h, mu_h, mi_h, gu_h, gi_h, wg_h,
            out_mu, out_mi, out_g,
            uidx_v, iidx_v, wg_v, mu_s, mi_s, gu_s, gi_s, gacc, gsem, wsem):
        wid = lax.axis_index("s") * 2 + lax.axis_index("c")
        pltpu.sync_copy(uidx_h.at[pl.ds(wid * _NCH, _NCH)], uidx_v)
        pltpu.sync_copy(iidx_h.at[pl.ds(wid * _NCH, _NCH)], iidx_v)

        g_desc = [None, None, None]
        w_desc = [None, None, None, None]

        def fire_gathers(j):
            s = j % 3
            g_desc[s] = [
                pltpu.async_copy(mu_h.at[uidx_v.at[j]], mu_s[j % 4], gsem[s]),
                pltpu.async_copy(mi_h.at[iidx_v.at[j]], mi_s[j % 4], gsem[s]),
                pltpu.async_copy(gu_h.at[uidx_v.at[j]], gu_s[s], gsem[s]),
                pltpu.async_copy(gi_h.at[iidx_v.at[j]], gi_s[s], gsem[s]),
            ]

        fire_gathers(0)
        fire_gathers(1)
        pltpu.sync_copy(wg_h.at[0, pl.ds(0, _D)], wg_v)
        for j in range(_NCH):
            ms = j % 3
            if j + 2 < _NCH:
                if j >= 2:
                    for d in w_desc[(j + 2) % 4]:
                        d.wait()
                fire_gathers(j + 2)
            for d in g_desc[ms]:
                d.wait()

            gur, gir = gu_s[ms], gi_s[ms]

            def gmf_group(g, _):
                row = g * _PK
                for k in range(_PK):
                    for c in range(_PK):
                        sl = pl.ds(c * _GL, _GL)
                        p = gur[row + k, sl] * gir[row + k, sl] * wg_v[sl]
                        acc = p if c == 0 else acc + p
                    gacc[j * (_CH // _PK) + g, pl.ds(k * _GL, _GL)] = acc
                return 0

            lax.fori_loop(0, _CH // _PK, gmf_group, 0)

            ob = wid * _BPW + j * _CH
            w_desc[j % 4] = [
                pltpu.async_copy(mu_s[j % 4], out_mu.at[pl.ds(ob, _CH)],
                                 wsem[j % 4]),
                pltpu.async_copy(mi_s[j % 4], out_mi.at[pl.ds(ob, _CH)],
                                 wsem[j % 4]),
            ]

        for j in range(_NCH - 4, _NCH):
            for d in w_desc[j % 4]:
                d.wait()
        pltpu.sync_copy(gacc, out_g.at[pl.ds(wid * (_BPW // _PK), _BPW // _PK)])

    return run(uidx2, iidx2, mu, mi, gu, gi, wg)


_BLK = 2048


def _tc_dense(um, im, gp, W1, b1, W2, b2, W3, b3, W_last, blast):
    f32 = jnp.float32

    def body(um_r, im_r, g_r, w1_r, b1_r, w2_r, b2_r, w3_r, b3_r,
             wl_r, bl_r, st_r, p_r, m_r, out_r):
        dot = functools.partial(lax.dot_general,
                                dimension_numbers=(((1,), (1,)), ((), ())),
                                preferred_element_type=f32)
        bf16 = jnp.bfloat16
        w1 = w1_r[...].astype(bf16)
        h = jnp.maximum(dot(um_r[...].astype(bf16), w1[:, :_D])
                        + dot(im_r[...].astype(bf16), w1[:, _D:])
                        + b1_r[...].reshape(1, 64), 0.0)
        h = jnp.maximum(dot(h.astype(bf16), w2_r[...].astype(bf16))
                        + b2_r[...].reshape(1, 16), 0.0)
        h = jnp.maximum(dot(h.astype(bf16), w3_r[...].astype(bf16))
                        + b3_r[...].reshape(1, 8), 0.0)

        rs = dot(g_r[...], st_r[...])
        ps = lax.dot_general(p_r[...], rs, (((1,), (0,)), ((), ())),
                             preferred_element_type=f32)
        g = jnp.sum(ps * m_r[...], axis=1, keepdims=True)

        wm = wl_r[...][:, _D:]
        out_r[...] = (g + jnp.sum(h * wm, axis=1, keepdims=True)
                      + bl_r[...].reshape(1, 1))

    iota = functools.partial(lax.broadcasted_iota, jnp.int32)
    bp = _BLK // _PK
    St = (iota((_PK, _D), 1) // _GL == iota((_PK, _D), 0)).astype(f32)
    P = (iota((_BLK, bp), 0) // _PK == iota((_BLK, bp), 1)).astype(f32)
    M = (iota((_BLK, _PK), 0) % _PK == iota((_BLK, _PK), 1)).astype(f32)

    full = lambda shape: pl.BlockSpec(shape, lambda i: (0,) * len(shape))
    batch = pl.BlockSpec((_BLK, _D), lambda i: (i, 0))
    return pl.pallas_call(
        body,
        grid=(_B // _BLK,),
        in_specs=[batch, batch,
                  pl.BlockSpec((_BLK // _PK, _D), lambda i: (i, 0)),
                  full((64, 2 * _D)), full((64,)),
                  full((16, 64)), full((16,)),
                  full((8, 16)), full((8,)),
                  full((1, _D + 8)), full((1,)),
                  full((_PK, _D)), full((_BLK, bp)), full((_BLK, _PK))],
        out_specs=pl.BlockSpec((_BLK, 1), lambda i: (i, 0)),
        out_shape=jax.ShapeDtypeStruct((_B, 1), f32),
    )(um, im, gp, W1, b1, W2, b2, W3, b3, W_last, blast, St, P, M)


def kernel(x, mlp_user_w, mlp_item_w, gmf_user_w, gmf_item_w,
           W1, b1, W2, b2, W3, b3, W_last, b_last):
    uidx2 = x[:, 0].reshape(_B // _CH, _CH)
    iidx2 = x[:, 1].reshape(_B // _CH, _CH)
    um, im, gp = _sc_gather_gmf(uidx2, iidx2,
                                mlp_user_w, mlp_item_w,
                                gmf_user_w, gmf_item_w, W_last)
    out = _tc_dense(um, im, gp, W1, b1, W2, b2, W3, b3, W_last, b_last)
    return out
